# baseline (speedup 1.0000x reference)
import functools

import jax
import jax.numpy as jnp
from jax import lax
from jax.experimental import pallas as pl
from jax.experimental.pallas import tpu as pltpu
from jax.experimental.pallas import tpu_sc as plsc

F = 26
V = 100000
D = 32
B = 4096
NC, NS, L = 2, 16, 16
NW = NC * NS
R = B * F
RPW = R // NW
CH = RPW // 128


def _embed_body(idx_hbm, offs_hbm, tab_hbm, out_hbm, idx_v, offs_v, rows_v, sem):
    wid = lax.axis_index("s") * NC + lax.axis_index("c")
    base = wid * CH

    pltpu.sync_copy(idx_hbm.at[wid], idx_v)
    pltpu.sync_copy(offs_hbm, offs_v)

    def add_offs(j, carry):
        for t in range(128 // L):
            sl = pl.ds(t * L, L)
            idx_v[j, sl] = idx_v[j, sl] + offs_v[j, sl]
        return carry

    lax.fori_loop(0, CH, add_offs, 0)

    copies = [
        pltpu.make_async_copy(tab_hbm.at[idx_v.at[j]], rows_v.at[j], sem)
        for j in range(CH)
    ]
    for c in copies:
        c.start()
    for c in copies:
        c.wait()

    pltpu.sync_copy(rows_v, out_hbm.at[pl.ds(base, CH)])


@jax.jit
def kernel(inputs, tables):
    idx = inputs.reshape(NW, CH, 128).astype(jnp.int32)
    offs = ((jnp.arange(RPW, dtype=jnp.int32) % F) * V).reshape(CH, 128)
    tab = tables.reshape(F * V, D)

    mesh = plsc.VectorSubcoreMesh(
        core_axis_name="c", subcore_axis_name="s", num_cores=NC, num_subcores=NS
    )
    out = pl.kernel(
        _embed_body,
        out_type=jax.ShapeDtypeStruct((R // 128, 128, D), jnp.float32),
        mesh=mesh,
        compiler_params=pltpu.CompilerParams(use_tc_tiling_on_sc=False),
        scratch_types=[
            pltpu.VMEM((CH, 128), jnp.int32),
            pltpu.VMEM((CH, 128), jnp.int32),
            pltpu.VMEM((CH, 128, D), jnp.float32),
            pltpu.SemaphoreType.DMA,
        ],
    )(idx, offs, tab)
    return out.reshape(B, F * D)


# speedup vs baseline: 5.4032x; 5.4032x over previous
import functools

import jax
import jax.numpy as jnp
from jax import lax
from jax.experimental import pallas as pl
from jax.experimental.pallas import tpu as pltpu
from jax.experimental.pallas import tpu_sc as plsc

F = 26
V = 100000
D = 32
B = 4096
NC, NS, L = 2, 16, 16
NW = NC * NS
NROW = F * D
CHUNKS = B // L


def _embed_body(idx_hbm, tab_hbm, out_hbm, idx_v, slice_v, out_v, sem_i, sem_t, sem_o):
    wid = lax.axis_index("s") * NC + lax.axis_index("c")

    for j in range(F):
        row = j * NW + wid
        cp_i = pltpu.make_async_copy(idx_hbm.at[j], idx_v, sem_i)
        cp_t = pltpu.make_async_copy(tab_hbm.at[row], slice_v, sem_t)
        cp_i.start()
        cp_t.start()
        if j > 0:
            pltpu.make_async_copy(out_hbm.at[(j - 1) * NW + wid], out_v, sem_o).wait()
        cp_i.wait()
        cp_t.wait()

        def gather_chunk(i, carry):
            sl = pl.ds(i * L, L)
            out_v[sl] = plsc.load_gather(slice_v, [idx_v[sl]])
            return carry

        lax.fori_loop(0, CHUNKS, gather_chunk, 0)
        pltpu.make_async_copy(out_v, out_hbm.at[row], sem_o).start()

    pltpu.make_async_copy(out_hbm.at[(F - 1) * NW + wid], out_v, sem_o).wait()


@jax.jit
def kernel(inputs, tables):
    idx = inputs.T.astype(jnp.int32)
    tab = tables.transpose(0, 2, 1).reshape(NROW, V)

    mesh = plsc.VectorSubcoreMesh(
        core_axis_name="c", subcore_axis_name="s", num_cores=NC, num_subcores=NS
    )
    out = pl.kernel(
        _embed_body,
        out_type=jax.ShapeDtypeStruct((NROW, B), jnp.float32),
        mesh=mesh,
        compiler_params=pltpu.CompilerParams(use_tc_tiling_on_sc=True, needs_layout_passes=False),
        scratch_types=[
            pltpu.VMEM((B,), jnp.int32),
            pltpu.VMEM((V,), jnp.float32),
            pltpu.VMEM((B,), jnp.float32),
            pltpu.SemaphoreType.DMA,
            pltpu.SemaphoreType.DMA,
            pltpu.SemaphoreType.DMA,
        ],
    )(idx, tab)
    return out.T.reshape(B, NROW)


# speedup vs baseline: 6.6280x; 1.2267x over previous
import functools

import jax
import jax.numpy as jnp
from jax import lax
from jax.experimental import pallas as pl
from jax.experimental.pallas import tpu as pltpu
from jax.experimental.pallas import tpu_sc as plsc

F = 26
V = 100000
D = 32
B = 4096
NC, NS, L = 2, 16, 16
NW = NC * NS
NROW = F * D
CHUNKS = B // L
UNROLL = 8


def _embed_body(idx_hbm, tab_hbm, out_hbm,
                idx0, idx1, slice_v, out_v, sem_i, sem_t, sem_o):
    wid = lax.axis_index("s") * NC + lax.axis_index("c")
    idx_bufs = (idx0, idx1)

    pltpu.make_async_copy(idx_hbm.at[0], idx0, sem_i).start()
    pltpu.make_async_copy(tab_hbm.at[wid], slice_v, sem_t).start()

    for j in range(F):
        row = j * NW + wid
        idx_v = idx_bufs[j % 2]
        pltpu.make_async_copy(tab_hbm.at[row], slice_v, sem_t).wait()
        if j + 1 < F:
            pltpu.make_async_copy(
                idx_hbm.at[j + 1], idx_bufs[(j + 1) % 2], sem_i).start()
        pltpu.make_async_copy(idx_hbm.at[j], idx_v, sem_i).wait()
        if j > 0:
            pltpu.make_async_copy(
                out_hbm.at[(j - 1) * NW + wid], out_v, sem_o).wait()

        def gather(i, carry):
            for t in range(UNROLL):
                sl = pl.ds((i * UNROLL + t) * L, L)
                out_v[sl] = plsc.load_gather(slice_v, [idx_v[sl]])
            return carry

        lax.fori_loop(0, CHUNKS // UNROLL, gather, 0)
        pltpu.make_async_copy(out_v, out_hbm.at[row], sem_o).start()
        if j + 1 < F:
            pltpu.make_async_copy(
                tab_hbm.at[(j + 1) * NW + wid], slice_v, sem_t).start()

    pltpu.make_async_copy(out_hbm.at[(F - 1) * NW + wid], out_v, sem_o).wait()


@jax.jit
def kernel(inputs, tables):
    idx = inputs.T.astype(jnp.int32)
    tab = tables.transpose(0, 2, 1).reshape(NROW, V)

    mesh = plsc.VectorSubcoreMesh(
        core_axis_name="c", subcore_axis_name="s", num_cores=NC, num_subcores=NS
    )
    out = pl.kernel(
        _embed_body,
        out_type=jax.ShapeDtypeStruct((NROW, B), jnp.float32),
        mesh=mesh,
        compiler_params=pltpu.CompilerParams(
            use_tc_tiling_on_sc=True, needs_layout_passes=False),
        scratch_types=[
            pltpu.VMEM((B,), jnp.int32),
            pltpu.VMEM((B,), jnp.int32),
            pltpu.VMEM((V,), jnp.float32),
            pltpu.VMEM((B,), jnp.float32),
            pltpu.SemaphoreType.DMA,
            pltpu.SemaphoreType.DMA,
            pltpu.SemaphoreType.DMA,
        ],
    )(idx, tab)
    return out.T.reshape(B, NROW)


# speedup vs baseline: 6.7265x; 1.0149x over previous
import functools

import jax
import jax.numpy as jnp
from jax import lax
from jax.experimental import pallas as pl
from jax.experimental.pallas import tpu as pltpu
from jax.experimental.pallas import tpu_sc as plsc

F = 26
V = 100000
D = 32
B = 4096
NC, NS, L = 2, 16, 16
NW = NC * NS
NROW = F * D
CHUNKS = B // L
UNROLL = 4
H0 = 50048
H1 = 49920
TV = 128
TH = V - TV


def _embed_body(idx_hbm, tab_hbm, tails_hbm, out_hbm,
                idx0, idx1, buf0, buf1, tails_v, out_v,
                sem_i, sem_h0, sem_h1, sem_tl, sem_o):
    wid = lax.axis_index("s") * NC + lax.axis_index("c")
    idx_bufs = (idx0, idx1)

    def h0_copy(j):
        return pltpu.make_async_copy(
            tab_hbm.at[j * NW + wid, pl.ds(0, H0)], buf0, sem_h0)

    def h1_copy(j):
        return pltpu.make_async_copy(
            tab_hbm.at[j * NW + wid, pl.ds(H0, H1)], buf1, sem_h1)

    h0_copy(0).start()
    h1_copy(0).start()
    pltpu.make_async_copy(idx_hbm.at[0], idx0, sem_i).start()
    for j in range(F):
        pltpu.make_async_copy(
            tails_hbm.at[j * NW + wid], tails_v.at[pl.ds(j * TV, TV)], sem_tl
        ).start()
    for j in range(F):
        pltpu.make_async_copy(
            tails_hbm.at[j * NW + wid], tails_v.at[pl.ds(j * TV, TV)], sem_tl
        ).wait()

    for j in range(F):
        row = j * NW + wid
        idx_v = idx_bufs[j % 2]
        h0_copy(j).wait()
        pltpu.make_async_copy(idx_hbm.at[j], idx_v, sem_i).wait()
        if j + 1 < F:
            pltpu.make_async_copy(
                idx_hbm.at[j + 1], idx_bufs[(j + 1) % 2], sem_i).start()
        if j > 0:
            pltpu.make_async_copy(
                out_hbm.at[(j - 1) * NW + wid], out_v, sem_o).wait()

        def pass0(i, carry):
            for t in range(UNROLL):
                sl = pl.ds((i * UNROLL + t) * L, L)
                out_v[sl] = plsc.load_gather(
                    buf0, [jnp.minimum(idx_v[sl], H0 - 1)])
            return carry

        lax.fori_loop(0, CHUNKS // UNROLL, pass0, 0)
        if j + 1 < F:
            h0_copy(j + 1).start()
        h1_copy(j).wait()

        tbase = j * TV - TH

        def pass1(i, carry):
            for t in range(UNROLL):
                sl = pl.ds((i * UNROLL + t) * L, L)
                ix = idx_v[sl]
                l1 = jnp.minimum(jnp.maximum(ix - H0, 0), H1 - 1)
                g1 = plsc.load_gather(buf1, [l1])
                l2 = jnp.maximum(ix + tbase, 0)
                g2 = plsc.load_gather(tails_v, [l2])
                r = jnp.where(ix >= H0, g1, out_v[sl])
                out_v[sl] = jnp.where(ix >= TH, g2, r)
            return carry

        lax.fori_loop(0, CHUNKS // UNROLL, pass1, 0)
        pltpu.make_async_copy(out_v, out_hbm.at[row], sem_o).start()
        if j + 1 < F:
            h1_copy(j + 1).start()

    pltpu.make_async_copy(out_hbm.at[(F - 1) * NW + wid], out_v, sem_o).wait()


@jax.jit
def kernel(inputs, tables):
    idx = inputs.T.astype(jnp.int32)
    tab = tables.transpose(0, 2, 1).reshape(NROW, V)
    tails = lax.slice(tab, (0, TH), (NROW, V))

    mesh = plsc.VectorSubcoreMesh(
        core_axis_name="c", subcore_axis_name="s", num_cores=NC, num_subcores=NS
    )
    out = pl.kernel(
        _embed_body,
        out_type=jax.ShapeDtypeStruct((NROW, B), jnp.float32),
        mesh=mesh,
        compiler_params=pltpu.CompilerParams(
            use_tc_tiling_on_sc=True, needs_layout_passes=False),
        scratch_types=[
            pltpu.VMEM((B,), jnp.int32),
            pltpu.VMEM((B,), jnp.int32),
            pltpu.VMEM((H0,), jnp.float32),
            pltpu.VMEM((H1,), jnp.float32),
            pltpu.VMEM((F * TV,), jnp.float32),
            pltpu.VMEM((B,), jnp.float32),
            pltpu.SemaphoreType.DMA,
            pltpu.SemaphoreType.DMA,
            pltpu.SemaphoreType.DMA,
            pltpu.SemaphoreType.DMA,
            pltpu.SemaphoreType.DMA,
        ],
    )(idx, tab, tails)
    return out.T.reshape(B, NROW)


# speedup vs baseline: 6.7430x; 1.0025x over previous
import jax
import jax.numpy as jnp
from jax import lax
from jax.experimental import pallas as pl
from jax.experimental.pallas import tpu as pltpu
from jax.experimental.pallas import tpu_sc as plsc

F = 26
V = 100000
D = 32
B = 4096
NC, NS, L = 2, 16, 16
NW = NC * NS
NROW = F * D
CHUNKS = B // L
UNROLL = 4
H0 = 50048
H1 = 49920
TV = 128
TH = V - TV


def _embed_body(idx_hbm, tab_hbm, tails_hbm, out_hbm,
                idx0, idx1, buf0, buf1, tails_v, out_v,
                sem_i, sem_h0, sem_h1, sem_tl, sem_o):
    wid = lax.axis_index("s") * NC + lax.axis_index("c")
    idx_bufs = (idx0, idx1)

    def h0_copy(j):
        return pltpu.make_async_copy(
            tab_hbm.at[j * NW + wid, pl.ds(0, H0)], buf0, sem_h0)

    def h1_copy(j):
        return pltpu.make_async_copy(
            tab_hbm.at[j * NW + wid, pl.ds(H0, H1)], buf1, sem_h1)

    h0_copy(0).start()
    h1_copy(0).start()
    pltpu.make_async_copy(idx_hbm.at[0], idx0, sem_i).start()
    for j in range(F):
        pltpu.make_async_copy(
            tails_hbm.at[j * NW + wid], tails_v.at[pl.ds(j * TV, TV)], sem_tl
        ).start()
    for j in range(F):
        pltpu.make_async_copy(
            tails_hbm.at[j * NW + wid], tails_v.at[pl.ds(j * TV, TV)], sem_tl
        ).wait()

    for j in range(F):
        row = j * NW + wid
        idx_v = idx_bufs[j % 2]
        h0_copy(j).wait()
        pltpu.make_async_copy(idx_hbm.at[j], idx_v, sem_i).wait()
        if j + 1 < F:
            pltpu.make_async_copy(
                idx_hbm.at[j + 1], idx_bufs[(j + 1) % 2], sem_i).start()
        if j > 0:
            pltpu.make_async_copy(
                out_hbm.at[(j - 1) * NW + wid], out_v, sem_o).wait()

        def pass0(i, carry):
            for t in range(UNROLL):
                sl = pl.ds((i * UNROLL + t) * L, L)
                out_v[sl] = plsc.load_gather(
                    buf0, [jnp.minimum(idx_v[sl], H0 - 1)])
            return carry

        lax.fori_loop(0, CHUNKS // UNROLL, pass0, 0)
        if j + 1 < F:
            h0_copy(j + 1).start()
        h1_copy(j).wait()

        tbase = j * TV - TH

        def pass1(i, carry):
            for t in range(UNROLL):
                sl = pl.ds((i * UNROLL + t) * L, L)
                ix = idx_v[sl]
                l1 = jnp.minimum(jnp.maximum(ix - H0, 0), H1 - 1)
                g1 = plsc.load_gather(buf1, [l1])
                l2 = jnp.maximum(ix + tbase, 0)
                g2 = plsc.load_gather(tails_v, [l2])
                r = jnp.where(ix >= H0, g1, out_v[sl])
                out_v[sl] = jnp.where(ix >= TH, g2, r)
            return carry

        lax.fori_loop(0, CHUNKS // UNROLL, pass1, 0)
        pltpu.make_async_copy(out_v, out_hbm.at[row], sem_o).start()
        if j + 1 < F:
            h1_copy(j + 1).start()

    pltpu.make_async_copy(out_hbm.at[(F - 1) * NW + wid], out_v, sem_o).wait()


@jax.jit
def kernel(inputs, tables):
    idx = inputs.T.astype(jnp.int32)
    tab = tables.transpose(0, 2, 1).reshape(NROW, V)
    tails = lax.slice(tab, (0, TH), (NROW, V))

    mesh = plsc.VectorSubcoreMesh(
        core_axis_name="c", subcore_axis_name="s", num_cores=NC, num_subcores=NS
    )
    out = pl.kernel(
        _embed_body,
        out_type=jax.ShapeDtypeStruct((NROW, B), jnp.float32),
        mesh=mesh,
        compiler_params=pltpu.CompilerParams(
            use_tc_tiling_on_sc=True, needs_layout_passes=False),
        scratch_types=[
            pltpu.VMEM((B,), jnp.int32),
            pltpu.VMEM((B,), jnp.int32),
            pltpu.VMEM((H0,), jnp.float32),
            pltpu.VMEM((H1,), jnp.float32),
            pltpu.VMEM((F * TV,), jnp.float32),
            pltpu.VMEM((B,), jnp.float32),
            pltpu.SemaphoreType.DMA,
            pltpu.SemaphoreType.DMA,
            pltpu.SemaphoreType.DMA,
            pltpu.SemaphoreType.DMA,
            pltpu.SemaphoreType.DMA,
        ],
    )(idx, tab, tails)
    return out.T.reshape(B, NROW)
